# baseline (speedup 1.0000x reference)
import functools

import jax
import jax.numpy as jnp
from jax import lax
from jax.experimental import pallas as pl
from jax.experimental.pallas import tpu as pltpu
from jax.experimental.pallas import tpu_sc as plsc

E = 8
K = 2
H = 2048
I = 768
T = 2048
PAIRS = T * K

BLK = 256
NBLK = PAIRS // BLK + E
PAD_T = NBLK * BLK

NC = 2
NS = 16
L = 16
NW = NC * NS
SLOTS_W = PAD_T // NW
TOK_W = T // NW
XCH = 16
CCH = 8


def _router_plan_body(x_ref, gw_ref, pos_ref, w_ref, bexp_ref, meta_ref):
    f32, i32 = jnp.float32, jnp.int32
    logits = jnp.dot(x_ref[...], gw_ref[...], preferred_element_type=f32)

    iota_e = lax.broadcasted_iota(i32, (T, E), 1)
    m1 = jnp.max(logits, axis=1, keepdims=True)
    a1 = jnp.min(jnp.where(logits == m1, iota_e, E), axis=1, keepdims=True)
    l2 = jnp.where(iota_e == a1, -jnp.inf, logits)
    m2 = jnp.max(l2, axis=1, keepdims=True)
    a2 = jnp.min(jnp.where(l2 == m2, iota_e, E), axis=1, keepdims=True)

    e2 = jnp.exp(m2 - m1)
    w1 = 1.0 / (1.0 + e2)
    w2 = e2 / (1.0 + e2)
    w_ref[...] = jnp.concatenate([w1, w2], axis=1)

    onehot0 = (iota_e == a1).astype(f32)
    onehot1 = (iota_e == a2).astype(f32)

    r_io = lax.broadcasted_iota(i32, (T, T), 0)
    c_io = lax.broadcasted_iota(i32, (T, T), 1)
    tri = (c_io <= r_io).astype(f32)
    c0 = jnp.dot(tri, onehot0, preferred_element_type=f32)
    c1 = jnp.dot(tri, onehot1, preferred_element_type=f32)
    c0i = c0.astype(i32)
    c1i = c1.astype(i32)
    cnt0 = c0i[T - 1:T, :]
    cnt1 = c1i[T - 1:T, :]
    cnt = cnt0 + cnt1
    nblk = (cnt + (BLK - 1)) // BLK
    meta_ref[...] = jnp.sum(nblk, axis=1, keepdims=True)

    er = lax.broadcasted_iota(i32, (E, E), 0)
    ec = lax.broadcasted_iota(i32, (E, E), 1)
    boff = jnp.sum(jnp.where(er < ec, jnp.broadcast_to(nblk.reshape(E, 1), (E, E)), 0),
                   axis=0, keepdims=True)
    padoff = boff * BLK

    mask0 = onehot0 > 0
    mask1 = onehot1 > 0
    pad0 = jnp.sum(jnp.where(mask0, padoff, 0), axis=1, keepdims=True)
    rank0 = jnp.sum(jnp.where(mask0, c0i, 0), axis=1, keepdims=True)
    pos0 = pad0 + rank0 - 1
    pad1 = jnp.sum(jnp.where(mask1, padoff, 0), axis=1, keepdims=True)
    base1 = jnp.sum(jnp.where(mask1, cnt0, 0), axis=1, keepdims=True)
    rank1 = jnp.sum(jnp.where(mask1, c1i, 0), axis=1, keepdims=True)
    pos1 = pad1 + base1 + rank1 - 1
    pos_ref[...] = jnp.concatenate([pos0, pos1], axis=1)

    b_io = lax.broadcasted_iota(i32, (NBLK, E), 0)
    ge = (b_io >= boff).astype(i32)
    bexp_ref[...] = jnp.sum(ge, axis=1, keepdims=True) - 1


def _router_plan(x, gate_w):
    return pl.pallas_call(
        _router_plan_body,
        out_shape=[
            jax.ShapeDtypeStruct((T, K), jnp.int32),
            jax.ShapeDtypeStruct((T, K), jnp.float32),
            jax.ShapeDtypeStruct((NBLK, 1), jnp.int32),
            jax.ShapeDtypeStruct((1, 1), jnp.int32),
        ],
    )(x, gate_w)


def _dispatch_body(x_hbm, posi_hbm, xs_hbm,
                   pv, idx_v, buf0, buf1, ls0, ls1, ss0, ss1):
    wid = lax.axis_index("s") * NC + lax.axis_index("c")
    tb = wid * TOK_W
    pltpu.sync_copy(posi_hbm.at[pl.ds(tb * K, K * TOK_W)], pv)

    lane = lax.iota(jnp.int32, L)
    nch = TOK_W // XCH
    for j in range(nch):
        idx_v[2 * j, :] = plsc.load_gather(pv, [j * 2 * XCH + 2 * lane])
        idx_v[2 * j + 1, :] = plsc.load_gather(pv, [j * 2 * XCH + 2 * lane + 1])

    bufs = (buf0, buf1)
    lsems = (ls0, ls1)
    ssems = (ss0, ss1)
    ld = [None, None]
    sc = [[None, None], [None, None]]

    def fire_load(j):
        p = j % 2
        ld[p] = pltpu.async_copy(x_hbm.at[pl.ds(tb + j * XCH, XCH)],
                                 bufs[p], lsems[p])

    fire_load(0)
    for j in range(nch):
        p = j % 2
        if j + 1 < nch:
            q = (j + 1) % 2
            for d in sc[q]:
                if d is not None:
                    d.wait()
            sc[q] = [None, None]
            fire_load(j + 1)
        ld[p].wait()
        sc[p][0] = pltpu.async_copy(bufs[p], xs_hbm.at[idx_v.at[2 * j]], ssems[p])
        sc[p][1] = pltpu.async_copy(bufs[p], xs_hbm.at[idx_v.at[2 * j + 1]], ssems[p])
    for p in range(2):
        for d in sc[p]:
            if d is not None:
                d.wait()


def _dispatch(x, posi):
    mesh = plsc.VectorSubcoreMesh(core_axis_name="c", subcore_axis_name="s",
                                  num_cores=NC, num_subcores=NS)
    kern = pl.kernel(
        _dispatch_body,
        out_type=jax.ShapeDtypeStruct((PAD_T, H // 2), jnp.int32),
        mesh=mesh,
        scratch_types=[
            pltpu.VMEM((K * TOK_W,), jnp.int32),
            pltpu.VMEM((2 * TOK_W // XCH, L), jnp.int32),
            pltpu.VMEM((XCH, H // 2), jnp.int32),
            pltpu.VMEM((XCH, H // 2), jnp.int32),
            pltpu.SemaphoreType.DMA,
            pltpu.SemaphoreType.DMA,
            pltpu.SemaphoreType.DMA,
            pltpu.SemaphoreType.DMA,
        ],
        compiler_params=pltpu.CompilerParams(needs_layout_passes=False),
    )
    return kern(x, posi)


def _ffn_body(bexp_ref, meta_ref, xs_ref, wg_ref, wu_ref, wd_ref, out_ref):
    b = pl.program_id(0)

    @pl.when(b < meta_ref[0])
    def _():
        xb = xs_ref[...].astype(jnp.float32)
        g = jnp.dot(xb, wg_ref[0], preferred_element_type=jnp.float32,
                    precision=lax.Precision.DEFAULT)
        u = jnp.dot(xb, wu_ref[0], preferred_element_type=jnp.float32,
                    precision=lax.Precision.DEFAULT)
        h = g * (1.0 / (1.0 + jnp.exp(-g))) * u
        d = jnp.dot(h, wd_ref[0], preferred_element_type=jnp.float32,
                    precision=lax.Precision.DEFAULT)
        out_ref[...] = d.astype(jnp.bfloat16)


def _expert_ffn(bexp, meta, xs, gate_proj_w, up_proj_w, down_proj_w):
    grid_spec = pltpu.PrefetchScalarGridSpec(
        num_scalar_prefetch=2,
        grid=(NBLK,),
        in_specs=[
            pl.BlockSpec((BLK, H), lambda b, be, mt: (jnp.minimum(b, mt[0] - 1), 0)),
            pl.BlockSpec((1, H, I), lambda b, be, mt: (be[b], 0, 0)),
            pl.BlockSpec((1, H, I), lambda b, be, mt: (be[b], 0, 0)),
            pl.BlockSpec((1, I, H), lambda b, be, mt: (be[b], 0, 0)),
        ],
        out_specs=pl.BlockSpec((BLK, H),
                               lambda b, be, mt: (jnp.minimum(b, mt[0] - 1), 0)),
    )
    return pl.pallas_call(
        _ffn_body,
        grid_spec=grid_spec,
        out_shape=jax.ShapeDtypeStruct((PAD_T, H), jnp.bfloat16),
        compiler_params=pltpu.CompilerParams(
            dimension_semantics=("arbitrary",)),
    )(bexp, meta, xs, gate_proj_w, up_proj_w, down_proj_w)


def _combine_body(ys_hbm, posi_hbm, wpi_hbm, out_hbm, pv, wv, p0_v, p1_v,
                  r0a, r1a, r0b, r1b, gsa, gsb, wsa, wsb):
    wid = lax.axis_index("s") * NC + lax.axis_index("c")
    tb = wid * TOK_W
    pltpu.sync_copy(posi_hbm.at[pl.ds(tb * K, K * TOK_W)], pv)
    pltpu.sync_copy(wpi_hbm.at[pl.ds(tb * K, K * TOK_W)], wv)

    lane = lax.iota(jnp.int32, L)
    for j in range(TOK_W // L):
        p0_v[pl.ds(j * L, L)] = plsc.load_gather(pv, [j * 2 * L + 2 * lane])
        p1_v[pl.ds(j * L, L)] = plsc.load_gather(pv, [j * 2 * L + 2 * lane + 1])

    nchunk = TOK_W // CCH
    r0s = (r0a, r0b)
    r1s = (r1a, r1b)
    gsems = (gsa, gsb)
    wsems = (wsa, wsb)
    gd = [None, None]
    wd = [None, None]

    def fire_gathers(c):
        p = c % 2
        s = pl.ds(c * CCH, CCH)
        d0 = pltpu.async_copy(ys_hbm.at[p0_v.at[s]], r0s[p], gsems[p])
        d1 = pltpu.async_copy(ys_hbm.at[p1_v.at[s]], r1s[p], gsems[p])
        gd[p] = (d0, d1)

    fire_gathers(0)
    for c in range(nchunk):
        p = c % 2
        if c + 1 < nchunk:
            if wd[1 - p] is not None:
                wd[1 - p].wait()
            fire_gathers(c + 1)
        gd[p][0].wait()
        gd[p][1].wait()
        r0, r1 = r0s[p], r1s[p]
        wc = wv[pl.ds(c * 2 * CCH, L)]
        for row in range(CCH):
            w0 = wc.at[jnp.full((L,), 2 * row, jnp.int32)].get(
                mode="promise_in_bounds")
            w1 = wc.at[jnp.full((L,), 2 * row + 1, jnp.int32)].get(
                mode="promise_in_bounds")
            w0b = plsc.pack(w0, w0, format=plsc.PackFormat.INTERLEAVED)
            w1b = plsc.pack(w1, w1, format=plsc.PackFormat.INTERLEAVED)

            def inner(j, _, row=row, w0b=w0b, w1b=w1b):
                for u in range(4):
                    s = pl.ds((j * 4 + u) * L, L)
                    v0 = plsc.bitcast(r0[row, s], jnp.bfloat16)
                    v1 = plsc.bitcast(r1[row, s], jnp.bfloat16)
                    r0[row, s] = plsc.bitcast(w0b * v0 + w1b * v1, jnp.int32)
                return 0
            lax.fori_loop(0, H // 2 // L // 4, inner, 0)
        wd[p] = pltpu.async_copy(r0, out_hbm.at[pl.ds(tb + c * CCH, CCH)],
                                 wsems[p])
    for p in range(2):
        if wd[p] is not None:
            wd[p].wait()


def _combine(ysw, posi, wpi):
    mesh = plsc.VectorSubcoreMesh(core_axis_name="c", subcore_axis_name="s",
                                  num_cores=NC, num_subcores=NS)
    kern = pl.kernel(
        _combine_body,
        out_type=jax.ShapeDtypeStruct((T, H // 2), jnp.int32),
        mesh=mesh,
        scratch_types=[
            pltpu.VMEM((K * TOK_W,), jnp.int32),
            pltpu.VMEM((K * TOK_W,), jnp.float32),
            pltpu.VMEM((TOK_W,), jnp.int32),
            pltpu.VMEM((TOK_W,), jnp.int32),
            pltpu.VMEM((CCH, H // 2), jnp.int32),
            pltpu.VMEM((CCH, H // 2), jnp.int32),
            pltpu.VMEM((CCH, H // 2), jnp.int32),
            pltpu.VMEM((CCH, H // 2), jnp.int32),
            pltpu.SemaphoreType.DMA,
            pltpu.SemaphoreType.DMA,
            pltpu.SemaphoreType.DMA,
            pltpu.SemaphoreType.DMA,
        ],
        compiler_params=pltpu.CompilerParams(needs_layout_passes=False),
    )
    return kern(ysw, posi, wpi)


def kernel(hidden_states, gate_w, gate_proj_w, up_proj_w, down_proj_w):
    B, S, Hh = hidden_states.shape
    x = hidden_states.reshape(S * B, Hh)

    pos2, w2, bexp2, meta2 = _router_plan(x, gate_w)

    posi = pos2.reshape(PAIRS)
    wpi = w2.reshape(PAIRS)

    xi = lax.bitcast_convert_type(
        x.astype(jnp.bfloat16).reshape(T, H // 2, 2), jnp.int32)
    xs_i = _dispatch(xi, posi)
    xs = lax.bitcast_convert_type(xs_i, jnp.bfloat16).reshape(PAD_T, H)

    ysw = _expert_ffn(bexp2.reshape(NBLK), meta2.reshape(1), xs,
                      gate_proj_w, up_proj_w, down_proj_w)

    ys_i = lax.bitcast_convert_type(ysw.reshape(PAD_T, H // 2, 2), jnp.int32)
    out_i = _combine(ys_i, posi, wpi)
    out = lax.bitcast_convert_type(out_i, jnp.bfloat16).reshape(T, H)
    return out.astype(jnp.float32).reshape(B, S, Hh)


# speedup vs baseline: 4.6342x; 4.6342x over previous
import functools

import jax
import jax.numpy as jnp
from jax import lax
from jax.experimental import pallas as pl
from jax.experimental.pallas import tpu as pltpu
from jax.experimental.pallas import tpu_sc as plsc

E = 8
K = 2
H = 2048
I = 768
T = 2048
PAIRS = T * K

BLK = 256
NBLK = PAIRS // BLK + E
PAD_T = NBLK * BLK

NC = 2
NS = 16
L = 16
NW = NC * NS
SLOTS_W = PAD_T // NW
TOK_W = T // NW
XCH = 16
CCH = 8


def _pack_cols(y16):
    hu = lax.bitcast_convert_type(y16[:, :H // 2], jnp.uint16).astype(jnp.uint32)
    lu = lax.bitcast_convert_type(y16[:, H // 2:], jnp.uint16).astype(jnp.uint32)
    return lax.bitcast_convert_type((hu << 16) | lu, jnp.int32)


def _unpack_cols(xi):
    xv = lax.bitcast_convert_type(xi, jnp.uint32)
    hi = lax.bitcast_convert_type((xv >> 16).astype(jnp.uint16), jnp.bfloat16)
    lo = lax.bitcast_convert_type(xv.astype(jnp.uint16), jnp.bfloat16)
    return jnp.concatenate([hi, lo], axis=1)


def _router_plan_body(x_ref, gw_ref, pos_ref, w_ref, bexp_ref, meta_ref, xi_ref):
    f32, i32 = jnp.float32, jnp.int32
    xi_ref[...] = _pack_cols(x_ref[...].astype(jnp.bfloat16))
    logits = jnp.dot(x_ref[...], gw_ref[...], preferred_element_type=f32)

    iota_e = lax.broadcasted_iota(i32, (T, E), 1)
    m1 = jnp.max(logits, axis=1, keepdims=True)
    a1 = jnp.min(jnp.where(logits == m1, iota_e, E), axis=1, keepdims=True)
    l2 = jnp.where(iota_e == a1, -jnp.inf, logits)
    m2 = jnp.max(l2, axis=1, keepdims=True)
    a2 = jnp.min(jnp.where(l2 == m2, iota_e, E), axis=1, keepdims=True)

    e2 = jnp.exp(m2 - m1)
    w1 = 1.0 / (1.0 + e2)
    w2 = e2 / (1.0 + e2)
    w_ref[...] = jnp.concatenate([w1, w2], axis=1)

    onehot0 = (iota_e == a1).astype(f32)
    onehot1 = (iota_e == a2).astype(f32)

    r_io = lax.broadcasted_iota(i32, (T, T), 0)
    c_io = lax.broadcasted_iota(i32, (T, T), 1)
    tri = (c_io <= r_io).astype(f32)
    c0 = jnp.dot(tri, onehot0, preferred_element_type=f32)
    c1 = jnp.dot(tri, onehot1, preferred_element_type=f32)
    c0i = c0.astype(i32)
    c1i = c1.astype(i32)
    cnt0 = c0i[T - 1:T, :]
    cnt1 = c1i[T - 1:T, :]
    cnt = cnt0 + cnt1
    nblk = (cnt + (BLK - 1)) // BLK
    meta_ref[...] = jnp.sum(nblk, axis=1, keepdims=True)

    er = lax.broadcasted_iota(i32, (E, E), 0)
    ec = lax.broadcasted_iota(i32, (E, E), 1)
    boff = jnp.sum(jnp.where(er < ec, jnp.broadcast_to(nblk.reshape(E, 1), (E, E)), 0),
                   axis=0, keepdims=True)
    padoff = boff * BLK

    mask0 = onehot0 > 0
    mask1 = onehot1 > 0
    pad0 = jnp.sum(jnp.where(mask0, padoff, 0), axis=1, keepdims=True)
    rank0 = jnp.sum(jnp.where(mask0, c0i, 0), axis=1, keepdims=True)
    pos0 = pad0 + rank0 - 1
    pad1 = jnp.sum(jnp.where(mask1, padoff, 0), axis=1, keepdims=True)
    base1 = jnp.sum(jnp.where(mask1, cnt0, 0), axis=1, keepdims=True)
    rank1 = jnp.sum(jnp.where(mask1, c1i, 0), axis=1, keepdims=True)
    pos1 = pad1 + base1 + rank1 - 1
    pos_ref[...] = jnp.concatenate([pos0, pos1], axis=1)

    b_io = lax.broadcasted_iota(i32, (NBLK, E), 0)
    ge = (b_io >= boff).astype(i32)
    bexp_ref[...] = jnp.sum(ge, axis=1, keepdims=True) - 1


def _router_plan(x, gate_w):
    return pl.pallas_call(
        _router_plan_body,
        out_shape=[
            jax.ShapeDtypeStruct((T, K), jnp.int32),
            jax.ShapeDtypeStruct((T, K), jnp.float32),
            jax.ShapeDtypeStruct((NBLK, 1), jnp.int32),
            jax.ShapeDtypeStruct((1, 1), jnp.int32),
            jax.ShapeDtypeStruct((T, H // 2), jnp.int32),
        ],
    )(x, gate_w)


def _dispatch_body(x_hbm, posi_hbm, xs_hbm,
                   pv, idx_v, buf0, buf1, ls0, ls1, ss0, ss1):
    wid = lax.axis_index("s") * NC + lax.axis_index("c")
    tb = wid * TOK_W
    pltpu.sync_copy(posi_hbm.at[pl.ds(tb * K, K * TOK_W)], pv)

    lane = lax.iota(jnp.int32, L)
    nch = TOK_W // XCH
    for j in range(nch):
        idx_v[2 * j, :] = plsc.load_gather(pv, [j * 2 * XCH + 2 * lane])
        idx_v[2 * j + 1, :] = plsc.load_gather(pv, [j * 2 * XCH + 2 * lane + 1])

    bufs = (buf0, buf1)
    lsems = (ls0, ls1)
    ssems = (ss0, ss1)
    ld = [None, None]
    sc = [[None, None], [None, None]]

    def fire_load(j):
        p = j % 2
        ld[p] = pltpu.async_copy(x_hbm.at[pl.ds(tb + j * XCH, XCH)],
                                 bufs[p], lsems[p])

    fire_load(0)
    for j in range(nch):
        p = j % 2
        if j + 1 < nch:
            q = (j + 1) % 2
            for d in sc[q]:
                if d is not None:
                    d.wait()
            sc[q] = [None, None]
            fire_load(j + 1)
        ld[p].wait()
        sc[p][0] = pltpu.async_copy(bufs[p], xs_hbm.at[idx_v.at[2 * j]], ssems[p])
        sc[p][1] = pltpu.async_copy(bufs[p], xs_hbm.at[idx_v.at[2 * j + 1]], ssems[p])
    for p in range(2):
        for d in sc[p]:
            if d is not None:
                d.wait()


def _dispatch(x, posi):
    mesh = plsc.VectorSubcoreMesh(core_axis_name="c", subcore_axis_name="s",
                                  num_cores=NC, num_subcores=NS)
    kern = pl.kernel(
        _dispatch_body,
        out_type=jax.ShapeDtypeStruct((PAD_T, H // 2), jnp.int32),
        mesh=mesh,
        scratch_types=[
            pltpu.VMEM((K * TOK_W,), jnp.int32),
            pltpu.VMEM((2 * TOK_W // XCH, L), jnp.int32),
            pltpu.VMEM((XCH, H // 2), jnp.int32),
            pltpu.VMEM((XCH, H // 2), jnp.int32),
            pltpu.SemaphoreType.DMA,
            pltpu.SemaphoreType.DMA,
            pltpu.SemaphoreType.DMA,
            pltpu.SemaphoreType.DMA,
        ],
        compiler_params=pltpu.CompilerParams(needs_layout_passes=False),
    )
    return kern(x, posi)


def _ffn_body(bexp_ref, meta_ref, xs_ref, wg_ref, wu_ref, wd_ref, out_ref):
    b = pl.program_id(0)

    @pl.when(b < meta_ref[0])
    def _():
        xb = _unpack_cols(xs_ref[...]).astype(jnp.float32)
        g = jnp.dot(xb, wg_ref[0], preferred_element_type=jnp.float32,
                    precision=lax.Precision.DEFAULT)
        u = jnp.dot(xb, wu_ref[0], preferred_element_type=jnp.float32,
                    precision=lax.Precision.DEFAULT)
        h = g * (1.0 / (1.0 + jnp.exp(-g))) * u
        d = jnp.dot(h, wd_ref[0], preferred_element_type=jnp.float32,
                    precision=lax.Precision.DEFAULT)
        out_ref[...] = _pack_cols(d.astype(jnp.bfloat16))


def _expert_ffn(bexp, meta, xs, gate_proj_w, up_proj_w, down_proj_w):
    grid_spec = pltpu.PrefetchScalarGridSpec(
        num_scalar_prefetch=2,
        grid=(NBLK,),
        in_specs=[
            pl.BlockSpec((BLK, H // 2),
                         lambda b, be, mt: (jnp.minimum(b, mt[0] - 1), 0)),
            pl.BlockSpec((1, H, I), lambda b, be, mt: (be[b], 0, 0)),
            pl.BlockSpec((1, H, I), lambda b, be, mt: (be[b], 0, 0)),
            pl.BlockSpec((1, I, H), lambda b, be, mt: (be[b], 0, 0)),
        ],
        out_specs=pl.BlockSpec((BLK, H // 2),
                               lambda b, be, mt: (jnp.minimum(b, mt[0] - 1), 0)),
    )
    return pl.pallas_call(
        _ffn_body,
        grid_spec=grid_spec,
        out_shape=jax.ShapeDtypeStruct((PAD_T, H // 2), jnp.int32),
        compiler_params=pltpu.CompilerParams(
            dimension_semantics=("arbitrary",)),
    )(bexp, meta, xs, gate_proj_w, up_proj_w, down_proj_w)


def _combine_body(ys_hbm, posi_hbm, wpi_hbm, out_hbm, pv, wv, p0_v, p1_v,
                  r0a, r1a, r0b, r1b, oa, ob, gsa, gsb, wsa, wsb):
    wid = lax.axis_index("s") * NC + lax.axis_index("c")
    tb = wid * TOK_W
    pltpu.sync_copy(posi_hbm.at[pl.ds(tb * K, K * TOK_W)], pv)
    pltpu.sync_copy(wpi_hbm.at[pl.ds(tb * K, K * TOK_W)], wv)

    lane = lax.iota(jnp.int32, L)
    for j in range(TOK_W // L):
        p0_v[pl.ds(j * L, L)] = plsc.load_gather(pv, [j * 2 * L + 2 * lane])
        p1_v[pl.ds(j * L, L)] = plsc.load_gather(pv, [j * 2 * L + 2 * lane + 1])

    nchunk = TOK_W // CCH
    r0s = (r0a, r0b)
    r1s = (r1a, r1b)
    os_ = (oa, ob)
    gsems = (gsa, gsb)
    wsems = (wsa, wsb)
    gd = [None, None]
    wd = [None, None]

    def fire_gathers(c):
        p = c % 2
        s = pl.ds(c * CCH, CCH)
        d0 = pltpu.async_copy(ys_hbm.at[p0_v.at[s]], r0s[p], gsems[p])
        d1 = pltpu.async_copy(ys_hbm.at[p1_v.at[s]], r1s[p], gsems[p])
        gd[p] = (d0, d1)

    fire_gathers(0)
    for c in range(nchunk):
        p = c % 2
        if c + 1 < nchunk:
            if wd[1 - p] is not None:
                wd[1 - p].wait()
            fire_gathers(c + 1)
        gd[p][0].wait()
        gd[p][1].wait()
        r0, r1 = r0s[p], r1s[p]
        o = os_[p]
        wc = wv[pl.ds(c * 2 * CCH, L)]
        for row in range(CCH):
            w0 = wc.at[jnp.full((L,), 2 * row, jnp.int32)].get(
                mode="promise_in_bounds")
            w1 = wc.at[jnp.full((L,), 2 * row + 1, jnp.int32)].get(
                mode="promise_in_bounds")
            w0b = plsc.pack(w0, w0, format=plsc.PackFormat.INTERLEAVED)
            w1b = plsc.pack(w1, w1, format=plsc.PackFormat.INTERLEAVED)

            def inner(j, _, row=row, w0b=w0b, w1b=w1b):
                for u in range(4):
                    k = j * 4 + u
                    s = pl.ds(k * L, L)
                    v0 = plsc.bitcast(r0[row, s], jnp.bfloat16)
                    v1 = plsc.bitcast(r1[row, s], jnp.bfloat16)
                    res = w0b * v0 + w1b * v1
                    lo_f, hi_f = plsc.unpack(
                        res, format=plsc.PackFormat.INTERLEAVED)
                    o[row, pl.ds(k * L, L)] = hi_f
                    o[row, pl.ds(H // 2 + k * L, L)] = lo_f
                return 0
            lax.fori_loop(0, H // 2 // L // 4, inner, 0)
        wd[p] = pltpu.async_copy(o, out_hbm.at[pl.ds(tb + c * CCH, CCH)],
                                 wsems[p])
    for p in range(2):
        if wd[p] is not None:
            wd[p].wait()


def _combine(ysw, posi, wpi):
    mesh = plsc.VectorSubcoreMesh(core_axis_name="c", subcore_axis_name="s",
                                  num_cores=NC, num_subcores=NS)
    kern = pl.kernel(
        _combine_body,
        out_type=jax.ShapeDtypeStruct((T, H), jnp.float32),
        mesh=mesh,
        scratch_types=[
            pltpu.VMEM((K * TOK_W,), jnp.int32),
            pltpu.VMEM((K * TOK_W,), jnp.float32),
            pltpu.VMEM((TOK_W,), jnp.int32),
            pltpu.VMEM((TOK_W,), jnp.int32),
            pltpu.VMEM((CCH, H // 2), jnp.int32),
            pltpu.VMEM((CCH, H // 2), jnp.int32),
            pltpu.VMEM((CCH, H // 2), jnp.int32),
            pltpu.VMEM((CCH, H // 2), jnp.int32),
            pltpu.VMEM((CCH, H), jnp.float32),
            pltpu.VMEM((CCH, H), jnp.float32),
            pltpu.SemaphoreType.DMA,
            pltpu.SemaphoreType.DMA,
            pltpu.SemaphoreType.DMA,
            pltpu.SemaphoreType.DMA,
        ],
        compiler_params=pltpu.CompilerParams(needs_layout_passes=False),
    )
    return kern(ysw, posi, wpi)


def kernel(hidden_states, gate_w, gate_proj_w, up_proj_w, down_proj_w):
    B, S, Hh = hidden_states.shape
    x = hidden_states.reshape(S * B, Hh)

    pos2, w2, bexp2, meta2, xi = _router_plan(x, gate_w)

    posi = pos2.reshape(PAIRS)
    wpi = w2.reshape(PAIRS)

    xs_i = _dispatch(xi, posi)

    ys_i = _expert_ffn(bexp2.reshape(NBLK), meta2.reshape(1), xs_i,
                       gate_proj_w, up_proj_w, down_proj_w)

    out = _combine(ys_i, posi, wpi)
    return out.reshape(B, S, Hh)
